# baseline (speedup 1.0000x reference)
import functools

import jax
import jax.numpy as jnp
from jax import lax
from jax.experimental import pallas as pl
from jax.experimental.pallas import tpu as pltpu
from jax.experimental.pallas import tpu_sc as plsc

B, N, DIM = 16, 256, 384
K = 8192
DIM_MEM = 256
M = B * N

BM = 512
BK = 1024
NH = 4
MB = M // BM
KB = K // BK


def _argmax_body(x_ref, cb_ref, f2_ref, n2_ref, out_ref,
                 mval_ref, midx_ref, ta_ref, tb_ref):
    k = pl.program_id(1)

    j = jnp.maximum(k - 1, 0)

    def _argmin(tprev_ref):
        t = tprev_ref[...]
        rowmin = jnp.min(t, axis=1, keepdims=True)
        iota = lax.broadcasted_iota(jnp.int32, (1, BK), 1).astype(jnp.float32)
        rowarg_f = jnp.min(
            jnp.where(t == rowmin, iota, jnp.float32(2**30)),
            axis=1, keepdims=True,
        )
        rowarg = rowarg_f.astype(jnp.int32) + j * BK

        better = (rowmin < mval_ref[...]) | (j == 0)
        mval_ref[...] = jnp.where(better, rowmin, mval_ref[...])
        midx_ref[...] = jnp.where(better, rowarg, midx_ref[...])
        out_ref[...] = midx_ref[...][None]

    def _step(tcur_ref, tprev_ref):
        mm2 = lax.dot_general(
            x_ref[...], cb_ref[pl.ds(k * BK, BK), :],
            (((1,), (1,)), ((), ())),
            preferred_element_type=jnp.float32,
            precision=lax.Precision.DEFAULT,
        )
        tcur_ref[...] = (f2_ref[...] - mm2) + n2_ref[0:1, pl.ds(k * BK, BK)]
        _argmin(tprev_ref)

    @pl.when(lax.rem(k, 2) == 0)
    def _even():
        @pl.when(k < KB)
        def _full():
            _step(ta_ref, tb_ref)

    @pl.when(lax.rem(k, 2) == 1)
    def _odd():
        @pl.when(k < KB)
        def _full():
            _step(tb_ref, ta_ref)

    @pl.when(k == KB)
    def _drain():
        _argmin(ta_ref if KB % 2 == 1 else tb_ref)


def _nearest_codes(flatten, codebook, interpret=False):
    x2 = flatten * 2.0
    f2 = jnp.sum(flatten ** 2, axis=1, keepdims=True)
    n2 = jnp.sum(codebook ** 2, axis=1)[None, :]
    out = pl.pallas_call(
        _argmax_body,
        grid=(MB, KB + 1),
        in_specs=[
            pl.BlockSpec((BM, DIM), lambda m, k: (m, 0)),
            pl.BlockSpec((K, DIM), lambda m, k: (0, 0)),
            pl.BlockSpec((BM, 1), lambda m, k: (m, 0)),
            pl.BlockSpec((1, K), lambda m, k: (0, 0)),
        ],
        out_specs=pl.BlockSpec((1, BM, 1), lambda m, k: (m, 0, 0)),
        out_shape=jax.ShapeDtypeStruct((MB, BM, 1), jnp.int32),
        scratch_shapes=[
            pltpu.VMEM((BM, 1), jnp.float32),
            pltpu.VMEM((BM, 1), jnp.int32),
            pltpu.VMEM((BM, BK), jnp.float32),
            pltpu.VMEM((BM, BK), jnp.float32),
        ],
        interpret=interpret,
    )(x2, codebook, f2, n2)
    return out.reshape(M)


@functools.cache
def _make_gather():
    info = plsc.get_sparse_core_info()
    nc, ns = info.num_cores, info.num_subcores
    rows_per_w = M // (nc * ns)

    def _gather_body(idx_hbm, values_hbm, out_hbm, idx_v, rows_v, sem):
        wid = lax.axis_index("s") * nc + lax.axis_index("c")
        base = wid * rows_per_w
        pltpu.sync_copy(idx_hbm.at[pl.ds(base, rows_per_w)], idx_v)
        pltpu.async_copy(values_hbm.at[idx_v], rows_v, sem).wait()
        pltpu.sync_copy(rows_v, out_hbm.at[pl.ds(base, rows_per_w)])

    return pl.kernel(
        _gather_body,
        out_type=jax.ShapeDtypeStruct((M, DIM_MEM), jnp.float32),
        mesh=plsc.VectorSubcoreMesh(core_axis_name="c", subcore_axis_name="s"),
        scratch_types=[
            pltpu.VMEM((rows_per_w,), jnp.int32),
            pltpu.VMEM((rows_per_w, DIM_MEM), jnp.float32),
            pltpu.SemaphoreType.DMA,
        ],
    )


def kernel(x, codebook, values):
    flatten = x.reshape(M, DIM)
    idx = _nearest_codes(flatten, codebook)
    memories = _make_gather()(idx, values)
    return memories.reshape(B, N, DIM_MEM)


# speedup vs baseline: 1.0142x; 1.0142x over previous
import functools

import jax
import jax.numpy as jnp
from jax import lax
from jax.experimental import pallas as pl
from jax.experimental.pallas import tpu as pltpu
from jax.experimental.pallas import tpu_sc as plsc

B, N, DIM = 16, 256, 384
K = 8192
DIM_MEM = 256
M = B * N

BM = 512
BK = 1024
NH = 4
MB = M // BM
KB = K // BK


def _argmax_body(x_ref, cb_ref, f2_ref, n2_ref, out_ref,
                 mval_ref, midx_ref, ta_ref, tb_ref):
    k = pl.program_id(1)

    j = jnp.maximum(k - 1, 0)

    def _argmin(tprev_ref, emit=False):
        t = tprev_ref[...]
        rowmin = jnp.min(t, axis=1, keepdims=True)
        iota = lax.broadcasted_iota(jnp.int32, (1, BK), 1).astype(jnp.float32)
        rowarg_f = jnp.min(
            jnp.where(t == rowmin, iota, jnp.float32(2**30)),
            axis=1, keepdims=True,
        )
        rowarg = rowarg_f.astype(jnp.int32) + j * BK

        better = (rowmin < mval_ref[...]) | (j == 0)
        mval_ref[...] = jnp.where(better, rowmin, mval_ref[...])
        midx_ref[...] = jnp.where(better, rowarg, midx_ref[...])
        if emit:
            out_ref[...] = midx_ref[...][None]

    def _step(tcur_ref, tprev_ref):
        mm2 = lax.dot_general(
            x_ref[...], cb_ref[pl.ds(k * BK, BK), :],
            (((1,), (1,)), ((), ())),
            preferred_element_type=jnp.float32,
            precision=lax.Precision.DEFAULT,
        )
        tcur_ref[...] = (f2_ref[...] - mm2) + n2_ref[0:1, pl.ds(k * BK, BK)]
        _argmin(tprev_ref)

    @pl.when(lax.rem(k, 2) == 0)
    def _even():
        @pl.when(k < KB)
        def _full():
            _step(ta_ref, tb_ref)

    @pl.when(lax.rem(k, 2) == 1)
    def _odd():
        @pl.when(k < KB)
        def _full():
            _step(tb_ref, ta_ref)

    @pl.when(k == KB)
    def _drain():
        _argmin(ta_ref if KB % 2 == 1 else tb_ref, emit=True)


def _nearest_codes(flatten, codebook, interpret=False):
    x2 = flatten * 2.0
    f2 = jnp.sum(flatten ** 2, axis=1, keepdims=True)
    n2 = jnp.sum(codebook ** 2, axis=1)[None, :]
    out = pl.pallas_call(
        _argmax_body,
        grid=(MB, KB + 1),
        in_specs=[
            pl.BlockSpec((BM, DIM), lambda m, k: (m, 0)),
            pl.BlockSpec((K, DIM), lambda m, k: (0, 0)),
            pl.BlockSpec((BM, 1), lambda m, k: (m, 0)),
            pl.BlockSpec((1, K), lambda m, k: (0, 0)),
        ],
        out_specs=pl.BlockSpec((1, BM, 1), lambda m, k: (m, 0, 0)),
        out_shape=jax.ShapeDtypeStruct((MB, BM, 1), jnp.int32),
        scratch_shapes=[
            pltpu.VMEM((BM, 1), jnp.float32),
            pltpu.VMEM((BM, 1), jnp.int32),
            pltpu.VMEM((BM, BK), jnp.float32),
            pltpu.VMEM((BM, BK), jnp.float32),
        ],
        interpret=interpret,
    )(x2, codebook, f2, n2)
    return out.reshape(M)


@functools.cache
def _make_gather():
    info = plsc.get_sparse_core_info()
    nc, ns = info.num_cores, info.num_subcores
    rows_per_w = M // (nc * ns)

    def _gather_body(idx_hbm, values_hbm, out_hbm, idx_v, rows_v, sem):
        wid = lax.axis_index("s") * nc + lax.axis_index("c")
        base = wid * rows_per_w
        pltpu.sync_copy(idx_hbm.at[pl.ds(base, rows_per_w)], idx_v)
        pltpu.async_copy(values_hbm.at[idx_v], rows_v, sem).wait()
        pltpu.sync_copy(rows_v, out_hbm.at[pl.ds(base, rows_per_w)])

    return pl.kernel(
        _gather_body,
        out_type=jax.ShapeDtypeStruct((M, DIM_MEM), jnp.float32),
        mesh=plsc.VectorSubcoreMesh(core_axis_name="c", subcore_axis_name="s"),
        scratch_types=[
            pltpu.VMEM((rows_per_w,), jnp.int32),
            pltpu.VMEM((rows_per_w, DIM_MEM), jnp.float32),
            pltpu.SemaphoreType.DMA,
        ],
    )


def kernel(x, codebook, values):
    flatten = x.reshape(M, DIM)
    idx = _nearest_codes(flatten, codebook)
    memories = _make_gather()(idx, values)
    return memories.reshape(B, N, DIM_MEM)


# speedup vs baseline: 1.2332x; 1.2159x over previous
import functools

import jax
import jax.numpy as jnp
from jax import lax
from jax.experimental import pallas as pl
from jax.experimental.pallas import tpu as pltpu
from jax.experimental.pallas import tpu_sc as plsc

B, N, DIM = 16, 256, 384
K = 8192
DIM_MEM = 256
M = B * N

BM = 512
BK = 1024
NH = 4
MB = M // BM
KB = K // BK


def _argmax_body(x_ref, cb_ref, f2_ref, n2_ref, out_ref):
    x_blk = x_ref[...]
    f2 = f2_ref[...]
    iota = lax.broadcasted_iota(jnp.int32, (1, BK), 1).astype(jnp.float32)
    mval = None
    midx = None
    for k in range(KB):
        mm2 = lax.dot_general(
            x_blk, cb_ref[pl.ds(k * BK, BK), :],
            (((1,), (1,)), ((), ())),
            preferred_element_type=jnp.float32,
            precision=lax.Precision.DEFAULT,
        )
        t = (f2 - mm2) + n2_ref[0:1, pl.ds(k * BK, BK)]
        rowmin = jnp.min(t, axis=1, keepdims=True)
        rowarg_f = jnp.min(
            jnp.where(t == rowmin, iota, jnp.float32(2**30)),
            axis=1, keepdims=True,
        )
        rowarg = rowarg_f.astype(jnp.int32) + k * BK
        if k == 0:
            mval, midx = rowmin, rowarg
        else:
            better = rowmin < mval
            mval = jnp.where(better, rowmin, mval)
            midx = jnp.where(better, rowarg, midx)
    out_ref[...] = midx[None]


def _nearest_codes(flatten, codebook, interpret=False):
    x2 = flatten * 2.0
    f2 = jnp.sum(flatten ** 2, axis=1, keepdims=True)
    n2 = jnp.sum(codebook ** 2, axis=1)[None, :]
    out = pl.pallas_call(
        _argmax_body,
        grid=(MB,),
        in_specs=[
            pl.BlockSpec((BM, DIM), lambda m: (m, 0)),
            pl.BlockSpec((K, DIM), lambda m: (0, 0)),
            pl.BlockSpec((BM, 1), lambda m: (m, 0)),
            pl.BlockSpec((1, K), lambda m: (0, 0)),
        ],
        out_specs=pl.BlockSpec((1, BM, 1), lambda m: (m, 0, 0)),
        out_shape=jax.ShapeDtypeStruct((MB, BM, 1), jnp.int32),
        interpret=interpret,
    )(x2, codebook, f2, n2)
    return out.reshape(M)


@functools.cache
def _make_gather():
    info = plsc.get_sparse_core_info()
    nc, ns = info.num_cores, info.num_subcores
    rows_per_w = M // (nc * ns)

    def _gather_body(idx_hbm, values_hbm, out_hbm, idx_v, rows_v, sem):
        wid = lax.axis_index("s") * nc + lax.axis_index("c")
        base = wid * rows_per_w
        pltpu.sync_copy(idx_hbm.at[pl.ds(base, rows_per_w)], idx_v)
        pltpu.async_copy(values_hbm.at[idx_v], rows_v, sem).wait()
        pltpu.sync_copy(rows_v, out_hbm.at[pl.ds(base, rows_per_w)])

    return pl.kernel(
        _gather_body,
        out_type=jax.ShapeDtypeStruct((M, DIM_MEM), jnp.float32),
        mesh=plsc.VectorSubcoreMesh(core_axis_name="c", subcore_axis_name="s"),
        scratch_types=[
            pltpu.VMEM((rows_per_w,), jnp.int32),
            pltpu.VMEM((rows_per_w, DIM_MEM), jnp.float32),
            pltpu.SemaphoreType.DMA,
        ],
    )


def kernel(x, codebook, values):
    flatten = x.reshape(M, DIM)
    idx = _nearest_codes(flatten, codebook)
    memories = _make_gather()(idx, values)
    return memories.reshape(B, N, DIM_MEM)
